# baseline (speedup 1.0000x reference)
import functools

import jax
import jax.numpy as jnp
from jax import lax
from jax.experimental import pallas as pl
from jax.experimental.pallas import tpu as pltpu
from jax.experimental.pallas import tpu_sc as plsc

_VOCAB = 100000
_D = 1024
_B = 4
_S = 8192
_SCALE = 32.0
_EPS = 1e-5

_L = 16
_NVEC = _D // _L
_R = _S * _B
_NW = 32
_RPW = _R // _NW
_C = 32
_NCH = _RPW // _C


def _rsqrt_v(x):
    i = lax.bitcast_convert_type(x, jnp.int32)
    i = jnp.int32(0x5F3759DF) - lax.shift_right_arithmetic(i, 1)
    y = lax.bitcast_convert_type(i, jnp.float32)
    for _ in range(3):
        y = y * (1.5 - 0.5 * x * y * y)
    return y


_GATHER_DNUMS = lax.GatherDimensionNumbers(
    offset_dims=(), collapsed_slice_dims=(0,), start_index_map=(0,))


def _lane_sum(s):
    io = lax.iota(jnp.int32, 16)
    for k in (8, 4, 2, 1):
        perm = lax.gather(s, (io ^ k)[:, None], _GATHER_DNUMS, (1,),
                          mode=lax.GatherScatterMode.PROMISE_IN_BOUNDS)
        s = s + perm
    return s


def _emb_body(idx_hbm, tok_hbm, pos_hbm, gam_hbm, bet_hbm, out_hbm,
              idx_v, rows_v, pos_v, gam_v, bet_v, sem):
    wid = lax.axis_index("s") * 2 + lax.axis_index("c")
    base = wid * _RPW

    pltpu.sync_copy(gam_hbm, gam_v)
    pltpu.sync_copy(bet_hbm, bet_v)

    def chunk_body(g, _):
        row0 = pl.multiple_of(base + g * _C, _C)
        pos0 = pl.multiple_of(row0 // _B, _C // _B)
        pltpu.sync_copy(idx_hbm.at[pl.ds(row0, _C)], idx_v)
        pltpu.async_copy(tok_hbm.at[idx_v], rows_v, sem).wait()
        pltpu.sync_copy(pos_hbm.at[pl.ds(pos0, _C // _B)], pos_v)

        def row_body(i, _):
            pi = i // _B

            def acc(j, carry):
                s, s2 = carry
                o = pl.ds(pl.multiple_of(j * _L, _L), _L)
                y = rows_v[i, o] * _SCALE + pos_v[pi, o]
                rows_v[i, o] = y
                return (s + y, s2 + y * y)

            zero = jnp.zeros((_L,), jnp.float32)
            s, s2 = lax.fori_loop(0, _NVEC, acc, (zero, zero))
            mean_v = _lane_sum(s) * (1.0 / _D)
            var_v = _lane_sum(s2) * (1.0 / _D) - mean_v * mean_v
            inv_v = _rsqrt_v(var_v + _EPS)

            def norm(j, _):
                o = pl.ds(pl.multiple_of(j * _L, _L), _L)
                y = (rows_v[i, o] - mean_v) * inv_v
                rows_v[i, o] = y * gam_v[o] + bet_v[o]
                return 0

            lax.fori_loop(0, _NVEC, norm, 0)
            return 0

        lax.fori_loop(0, _C, row_body, 0)
        pltpu.sync_copy(rows_v, out_hbm.at[pl.ds(row0, _C)])
        return 0

    lax.fori_loop(0, _NCH, chunk_body, 0)


_emb_kernel = functools.partial(
    pl.kernel,
    mesh=plsc.VectorSubcoreMesh(core_axis_name="c", subcore_axis_name="s"),
    out_type=jax.ShapeDtypeStruct((_R, _D), jnp.float32),
    scratch_types=[
        pltpu.VMEM((_C,), jnp.int32),
        pltpu.VMEM((_C, _D), jnp.float32),
        pltpu.VMEM((_C // _B, _D), jnp.float32),
        pltpu.VMEM((_D,), jnp.float32),
        pltpu.VMEM((_D,), jnp.float32),
        pltpu.SemaphoreType.DMA,
    ],
)(_emb_body)


def kernel(src_tokens, tok_table, pos_table, ln_gamma, ln_beta):
    idx = src_tokens.T.reshape(-1)
    out = _emb_kernel(idx, tok_table, pos_table, ln_gamma, ln_beta)
    return out.reshape(_S, _B, _D)


# speedup vs baseline: 2.3322x; 2.3322x over previous
import functools

import jax
import jax.numpy as jnp
from jax import lax
from jax.experimental import pallas as pl
from jax.experimental.pallas import tpu as pltpu
from jax.experimental.pallas import tpu_sc as plsc

_VOCAB = 100000
_D = 1024
_B = 4
_S = 8192
_SCALE = 32.0
_EPS = 1e-5

_L = 16
_NVEC = _D // _L
_R = _S * _B
_NW = 32
_RPW = _R // _NW
_C = 32
_NCH = _RPW // _C
_U = 8


def _rsqrt_v(x):
    i = lax.bitcast_convert_type(x, jnp.int32)
    i = jnp.int32(0x5F3759DF) - lax.shift_right_arithmetic(i, 1)
    y = lax.bitcast_convert_type(i, jnp.float32)
    for _ in range(3):
        y = y * (1.5 - 0.5 * x * y * y)
    return y


_GATHER_DNUMS = lax.GatherDimensionNumbers(
    offset_dims=(), collapsed_slice_dims=(0,), start_index_map=(0,))


def _lane_sum(s):
    io = lax.iota(jnp.int32, 16)
    for k in (8, 4, 2, 1):
        perm = lax.gather(s, (io ^ k)[:, None], _GATHER_DNUMS, (1,),
                          mode=lax.GatherScatterMode.PROMISE_IN_BOUNDS)
        s = s + perm
    return s


def _emb_body(idx_hbm, tok_hbm, pos_hbm, gam_hbm, bet_hbm, out_hbm,
              idx_v, rows_v, pos_v, gam_v, bet_v, sem):
    wid = lax.axis_index("s") * 2 + lax.axis_index("c")
    base = wid * _RPW

    pltpu.sync_copy(gam_hbm, gam_v)
    pltpu.sync_copy(bet_hbm, bet_v)

    def chunk_body(g, _):
        row0 = pl.multiple_of(base + g * _C, _C)
        pos0 = pl.multiple_of(row0 // _B, _C // _B)
        pltpu.sync_copy(idx_hbm.at[pl.ds(row0, _C)], idx_v)
        pltpu.async_copy(tok_hbm.at[idx_v], rows_v, sem).wait()
        pltpu.sync_copy(pos_hbm.at[pl.ds(pos0, _C // _B)], pos_v)

        def quad_body(q, _):
            i0 = q * _B

            def acc(jj, carry):
                ss = list(carry)
                for u in range(_U):
                    o = pl.ds(pl.multiple_of(jj * (_U * _L) + u * _L, _L), _L)
                    p = pos_v[q, o]
                    for r in range(_B):
                        y = rows_v[i0 + r, o] * _SCALE + p
                        rows_v[i0 + r, o] = y
                        ss[r] = ss[r] + y
                        ss[_B + r] = ss[_B + r] + y * y
                return tuple(ss)

            zero = jnp.zeros((_L,), jnp.float32)
            carry = lax.fori_loop(0, _NVEC // _U, acc, (zero,) * (2 * _B))
            means = [_lane_sum(carry[r]) * (1.0 / _D) for r in range(_B)]
            invs = [
                _rsqrt_v(_lane_sum(carry[_B + r]) * (1.0 / _D)
                         - means[r] * means[r] + _EPS)
                for r in range(_B)
            ]

            def norm(jj, _):
                for u in range(_U):
                    o = pl.ds(pl.multiple_of(jj * (_U * _L) + u * _L, _L), _L)
                    gmm = gam_v[o]
                    bt = bet_v[o]
                    for r in range(_B):
                        y = (rows_v[i0 + r, o] - means[r]) * invs[r]
                        rows_v[i0 + r, o] = y * gmm + bt
                return 0

            lax.fori_loop(0, _NVEC // _U, norm, 0)
            return 0

        lax.fori_loop(0, _C // _B, quad_body, 0)
        pltpu.sync_copy(rows_v, out_hbm.at[pl.ds(row0, _C)])
        return 0

    lax.fori_loop(0, _NCH, chunk_body, 0)


_emb_kernel = functools.partial(
    pl.kernel,
    mesh=plsc.VectorSubcoreMesh(core_axis_name="c", subcore_axis_name="s"),
    out_type=jax.ShapeDtypeStruct((_R, _D), jnp.float32),
    scratch_types=[
        pltpu.VMEM((_C,), jnp.int32),
        pltpu.VMEM((_C, _D), jnp.float32),
        pltpu.VMEM((_C // _B, _D), jnp.float32),
        pltpu.VMEM((_D,), jnp.float32),
        pltpu.VMEM((_D,), jnp.float32),
        pltpu.SemaphoreType.DMA,
    ],
)(_emb_body)


def kernel(src_tokens, tok_table, pos_table, ln_gamma, ln_beta):
    idx = src_tokens.T.reshape(-1)
    out = _emb_kernel(idx, tok_table, pos_table, ln_gamma, ln_beta)
    return out.reshape(_S, _B, _D)
